# Initial kernel scaffold; baseline (speedup 1.0000x reference)
import jax
import jax.numpy as jnp
from jax.experimental import pallas as pl


def kernel(x, embed_table, lin_w, lin_b):
    raise NotImplementedError("write your pallas kernel here")


# speedup vs baseline: 85.4186x; 85.4186x over previous
import functools

import jax
import jax.numpy as jnp
from jax import lax
from jax.experimental import pallas as pl
from jax.experimental.pallas import tpu as pltpu
from jax.experimental.pallas import tpu_sc as plsc

_B, _L = 16384, 200
_N = _B * _L
_V, _D = 100, 10
_VG = 7
_TPAD = _VG * 16 * _D

_INFO = plsc.get_sparse_core_info()
_NC, _NS = _INFO.num_cores, _INFO.num_subcores
_NW = _NC * _NS
_PER_W = _N // _NW
_CHUNK = 10240
_NCHUNK = _PER_W // _CHUNK


def _body(x_hbm, tab_hbm, w_hbm, out_hbm,
          tab_v, w_v, scores_v, idx_v, res_v):
    wid = lax.axis_index("s") * _NC + lax.axis_index("c")
    base = wid * _PER_W

    pltpu.sync_copy(tab_hbm, tab_v)
    pltpu.sync_copy(w_hbm, w_v)
    b_vec = w_v[pl.ds(_D * 16, 16)]

    for g in range(_VG):
        vid = lax.iota(jnp.int32, 16) + (g * 16)
        acc = b_vec
        for d in range(_D):
            col = plsc.load_gather(tab_v, [vid * _D + d])
            acc = acc + col * w_v[pl.ds(d * 16, 16)]
        scores_v[pl.ds(g * 16, 16)] = acc

    for g in range(_NCHUNK):
        off = base + g * _CHUNK
        pltpu.sync_copy(x_hbm.at[pl.ds(off, _CHUNK)], idx_v)

        def _gather(i, c):
            idx = idx_v[pl.ds(i * 16, 16)]
            res_v[pl.ds(i * 16, 16)] = plsc.load_gather(scores_v, [idx])
            return c
        lax.fori_loop(0, _CHUNK // 16, _gather, 0)

        pltpu.sync_copy(res_v, out_hbm.at[pl.ds(off, _CHUNK)])


@functools.partial(jax.jit, static_argnames=())
def _run(xf, tab_flat, wb):
    mesh = plsc.VectorSubcoreMesh(core_axis_name="c", subcore_axis_name="s")
    kfn = pl.kernel(
        _body,
        out_type=jax.ShapeDtypeStruct((_N,), jnp.float32),
        mesh=mesh,
        compiler_params=pltpu.CompilerParams(needs_layout_passes=False),
        scratch_types=[
            pltpu.VMEM((_TPAD,), jnp.float32),
            pltpu.VMEM(((_D + 1) * 16,), jnp.float32),
            pltpu.VMEM((_VG * 16,), jnp.float32),
            pltpu.VMEM((_CHUNK,), jnp.int32),
            pltpu.VMEM((_CHUNK,), jnp.float32),
        ],
    )
    return kfn(xf, tab_flat, wb)


def kernel(x, embed_table, lin_w, lin_b):
    xf = x.reshape(-1).astype(jnp.int32)
    tab_flat = jnp.pad(embed_table.reshape(-1), (0, _TPAD - _V * _D))
    wb = jnp.concatenate([
        jnp.repeat(lin_w.reshape(-1), 16),
        jnp.broadcast_to(lin_b, (16,)),
    ])
    out = _run(xf, tab_flat, wb)
    return out.reshape(_B, _L, 1)


# speedup vs baseline: 106.1340x; 1.2425x over previous
import functools

import jax
import jax.numpy as jnp
from jax import lax
from jax.experimental import pallas as pl
from jax.experimental.pallas import tpu as pltpu
from jax.experimental.pallas import tpu_sc as plsc

_B, _L = 16384, 200
_N = _B * _L
_V, _D = 100, 10
_VG = 7
_TPAD = _VG * 16 * _D

_INFO = plsc.get_sparse_core_info()
_NC, _NS = _INFO.num_cores, _INFO.num_subcores
_NW = _NC * _NS
_PER_W = _N // _NW
_CHUNK = 10240
_NCHUNK = _PER_W // _CHUNK


def _body(x_hbm, tab_hbm, w_hbm, out_hbm,
          tab_v, w_v, scores_v, idx_v, res_v):
    wid = lax.axis_index("s") * _NC + lax.axis_index("c")
    base = wid * _PER_W

    pltpu.sync_copy(tab_hbm, tab_v)
    pltpu.sync_copy(w_hbm, w_v)
    b_vec = w_v[pl.ds(_D * 16, 16)]

    for g in range(_VG):
        vid = lax.iota(jnp.int32, 16) + (g * 16)
        acc = b_vec
        for d in range(_D):
            col = plsc.load_gather(tab_v, [vid * _D + d])
            acc = acc + col * w_v[pl.ds(d * 16, 16)]
        scores_v[pl.ds(g * 16, 16)] = acc

    for g in range(_NCHUNK):
        off = base + g * _CHUNK
        pltpu.sync_copy(x_hbm.at[pl.ds(off, _CHUNK)], idx_v)

        @plsc.parallel_loop(0, _CHUNK, step=16, unroll=8)
        def _gather(i):
            idx = idx_v[pl.ds(i, 16)]
            res_v[pl.ds(i, 16)] = plsc.load_gather(scores_v, [idx])

        pltpu.sync_copy(res_v, out_hbm.at[pl.ds(off, _CHUNK)])


@functools.partial(jax.jit, static_argnames=())
def _run(xf, tab_flat, wb):
    mesh = plsc.VectorSubcoreMesh(core_axis_name="c", subcore_axis_name="s")
    kfn = pl.kernel(
        _body,
        out_type=jax.ShapeDtypeStruct((_N,), jnp.float32),
        mesh=mesh,
        compiler_params=pltpu.CompilerParams(needs_layout_passes=False),
        scratch_types=[
            pltpu.VMEM((_TPAD,), jnp.float32),
            pltpu.VMEM(((_D + 1) * 16,), jnp.float32),
            pltpu.VMEM((_VG * 16,), jnp.float32),
            pltpu.VMEM((_CHUNK,), jnp.int32),
            pltpu.VMEM((_CHUNK,), jnp.float32),
        ],
    )
    return kfn(xf, tab_flat, wb)


def kernel(x, embed_table, lin_w, lin_b):
    xf = x.reshape(-1).astype(jnp.int32)
    tab_flat = jnp.pad(embed_table.reshape(-1), (0, _TPAD - _V * _D))
    wb = jnp.concatenate([
        jnp.repeat(lin_w.reshape(-1), 16),
        jnp.broadcast_to(lin_b, (16,)),
    ])
    out = _run(xf, tab_flat, wb)
    return out.reshape(_B, _L, 1)


# speedup vs baseline: 116.9636x; 1.1020x over previous
import functools

import jax
import jax.numpy as jnp
from jax import lax
from jax.experimental import pallas as pl
from jax.experimental.pallas import tpu as pltpu
from jax.experimental.pallas import tpu_sc as plsc

_B, _L = 16384, 200
_N = _B * _L
_V, _D = 100, 10
_VG = 7
_TPAD = _VG * 16 * _D

_INFO = plsc.get_sparse_core_info()
_NC, _NS = _INFO.num_cores, _INFO.num_subcores
_NW = _NC * _NS
_PER_W = _N // _NW
_CHUNK = 12800
_NCHUNK = _PER_W // _CHUNK


def _body(x_hbm, tab_hbm, w_hbm, out_hbm,
          tab_v, w_v, scores_v, idx0_v, idx1_v, res0_v, res1_v,
          in0_sem, in1_sem, out0_sem, out1_sem):
    wid = lax.axis_index("s") * _NC + lax.axis_index("c")
    base = wid * _PER_W

    pltpu.sync_copy(tab_hbm, tab_v)
    pltpu.sync_copy(w_hbm, w_v)
    b_vec = w_v[pl.ds(_D * 16, 16)]

    for g in range(_VG):
        vid = lax.iota(jnp.int32, 16) + (g * 16)
        acc = b_vec
        for d in range(_D):
            col = plsc.load_gather(tab_v, [vid * _D + d])
            acc = acc + col * w_v[pl.ds(d * 16, 16)]
        scores_v[pl.ds(g * 16, 16)] = acc

    idx_bufs = (idx0_v, idx1_v)
    res_bufs = (res0_v, res1_v)
    in_sems = (in0_sem, in1_sem)
    out_sems = (out0_sem, out1_sem)

    pltpu.async_copy(x_hbm.at[pl.ds(base, _CHUNK)], idx_bufs[0], in_sems[0])
    for g in range(_NCHUNK):
        bi = g % 2
        off = base + g * _CHUNK
        if g + 1 < _NCHUNK:
            pltpu.async_copy(
                x_hbm.at[pl.ds(off + _CHUNK, _CHUNK)],
                idx_bufs[1 - bi], in_sems[1 - bi])
        pltpu.make_async_copy(
            x_hbm.at[pl.ds(off, _CHUNK)], idx_bufs[bi], in_sems[bi]).wait()
        if g >= 2:
            pltpu.make_async_copy(
                res_bufs[bi], out_hbm.at[pl.ds(off - 2 * _CHUNK, _CHUNK)],
                out_sems[bi]).wait()
        idx_v = idx_bufs[bi]
        res_v = res_bufs[bi]

        @plsc.parallel_loop(0, _CHUNK, step=16, unroll=8)
        def _gather(i):
            idx = idx_v[pl.ds(i, 16)]
            res_v[pl.ds(i, 16)] = plsc.load_gather(scores_v, [idx])

        pltpu.async_copy(res_v, out_hbm.at[pl.ds(off, _CHUNK)], out_sems[bi])

    for g in range(max(_NCHUNK - 2, 0), _NCHUNK):
        bi = g % 2
        off = base + g * _CHUNK
        pltpu.make_async_copy(
            res_bufs[bi], out_hbm.at[pl.ds(off, _CHUNK)], out_sems[bi]).wait()


@functools.partial(jax.jit, static_argnames=())
def _run(xf, tab_flat, wb):
    mesh = plsc.VectorSubcoreMesh(core_axis_name="c", subcore_axis_name="s")
    kfn = pl.kernel(
        _body,
        out_type=jax.ShapeDtypeStruct((_N,), jnp.float32),
        mesh=mesh,
        compiler_params=pltpu.CompilerParams(needs_layout_passes=False),
        scratch_types=[
            pltpu.VMEM((_TPAD,), jnp.float32),
            pltpu.VMEM(((_D + 1) * 16,), jnp.float32),
            pltpu.VMEM((_VG * 16,), jnp.float32),
            pltpu.VMEM((_CHUNK,), jnp.int32),
            pltpu.VMEM((_CHUNK,), jnp.int32),
            pltpu.VMEM((_CHUNK,), jnp.float32),
            pltpu.VMEM((_CHUNK,), jnp.float32),
            pltpu.SemaphoreType.DMA,
            pltpu.SemaphoreType.DMA,
            pltpu.SemaphoreType.DMA,
            pltpu.SemaphoreType.DMA,
        ],
    )
    return kfn(xf, tab_flat, wb)


def kernel(x, embed_table, lin_w, lin_b):
    xf = x.reshape(-1).astype(jnp.int32)
    tab_flat = jnp.pad(embed_table.reshape(-1), (0, _TPAD - _V * _D))
    wb = jnp.concatenate([
        jnp.repeat(lin_w.reshape(-1), 16),
        jnp.broadcast_to(lin_b, (16,)),
    ])
    out = _run(xf, tab_flat, wb)
    return out.reshape(_B, _L, 1)


# speedup vs baseline: 320.7239x; 2.7421x over previous
import functools

import jax
import jax.numpy as jnp
from jax import lax
from jax.experimental import pallas as pl
from jax.experimental.pallas import tpu as pltpu
from jax.experimental.pallas import tpu_sc as plsc

_B, _L = 16384, 200
_N = _B * _L
_V, _D = 100, 10
_VG = 7
_TPAD = _VG * 16 * _D

_INFO = plsc.get_sparse_core_info()
_NC, _NS = _INFO.num_cores, _INFO.num_subcores
_NW = _NC * _NS
_TR = _L // 8
_TC = _B // 128
_UNITS = 224
_NROUND = _UNITS // _NW


def _lut(tab_v, w_v, scores_v):
    b_vec = w_v[pl.ds(_D * 16, 16)]
    for g in range(_VG):
        vid = lax.iota(jnp.int32, 16) + (g * 16)
        acc = b_vec
        for d in range(_D):
            col = plsc.load_gather(tab_v, [vid * _D + d])
            acc = acc + col * w_v[pl.ds(d * 16, 16)]
        scores_v[pl.ds(g * 16, 16)] = acc


def _body(xq_hbm, tab_hbm, w_hbm, out_hbm,
          tab_v, w_v, scores_v, idx0_v, idx1_v, res0_v, res1_v,
          in0_sem, in1_sem, out0_sem, out1_sem):
    t = lax.axis_index("s") * _NC + lax.axis_index("c")

    pltpu.sync_copy(tab_hbm, tab_v)
    pltpu.sync_copy(w_hbm, w_v)
    _lut(tab_v, w_v, scores_v)

    idx_bufs = (idx0_v, idx1_v)
    res_bufs = (res0_v, res1_v)
    in_sems = (in0_sem, in1_sem)
    out_sems = (out0_sem, out1_sem)

    def in_cp(k, bi):
        c = jnp.minimum(t + _NW * k, _L - 1)
        return pltpu.make_async_copy(
            xq_hbm.at[c // 8, :, c % 8, :], idx_bufs[bi], in_sems[bi])

    def out_cp(k, bi):
        c = t + _NW * k
        return pltpu.make_async_copy(
            res_bufs[bi], out_hbm.at[jnp.minimum(c, _L - 1)], out_sems[bi])

    in_cp(0, 0).start()
    for k in range(_NROUND):
        bi = k % 2
        if k + 1 < _NROUND:
            in_cp(k + 1, 1 - bi).start()
        in_cp(k, bi).wait()
        if k >= 2:
            out_cp(k - 2, bi).wait()
        idx_v = idx_bufs[bi]
        res_v = res_bufs[bi]

        @plsc.parallel_loop(0, _TC, step=1, unroll=2)
        def _gather(j):
            for l in range(8):
                idx = idx_v[j, pl.ds(l * 16, 16)]
                res_v[j, pl.ds(l * 16, 16)] = plsc.load_gather(scores_v, [idx])

        @pl.when(t + _NW * k < _L)
        def _():
            out_cp(k, bi).start()

    for k in range(_NROUND - 2, _NROUND):
        @pl.when(t + _NW * k < _L)
        def _():
            out_cp(k, k % 2).wait()


@jax.jit
def _run(xq, tab_flat, wb):
    mesh = plsc.VectorSubcoreMesh(core_axis_name="c", subcore_axis_name="s")
    kfn = pl.kernel(
        _body,
        out_type=jax.ShapeDtypeStruct((_L, _TC, 128), jnp.float32),
        mesh=mesh,
        compiler_params=pltpu.CompilerParams(needs_layout_passes=False),
        scratch_types=[
            pltpu.VMEM((_TPAD,), jnp.float32),
            pltpu.VMEM(((_D + 1) * 16,), jnp.float32),
            pltpu.VMEM((_VG * 16,), jnp.float32),
            pltpu.VMEM((_TC, 128), jnp.int32),
            pltpu.VMEM((_TC, 128), jnp.int32),
            pltpu.VMEM((_TC, 128), jnp.float32),
            pltpu.VMEM((_TC, 128), jnp.float32),
            pltpu.SemaphoreType.DMA,
            pltpu.SemaphoreType.DMA,
            pltpu.SemaphoreType.DMA,
            pltpu.SemaphoreType.DMA,
        ],
    )
    return kfn(xq, tab_flat, wb)


def kernel(x, embed_table, lin_w, lin_b):
    xq = (x.astype(jnp.int32).T
          .reshape(_TR, 8, _TC, 128)
          .transpose(0, 2, 1, 3))
    tab_flat = jnp.pad(embed_table.reshape(-1), (0, _TPAD - _V * _D))
    wb = jnp.concatenate([
        jnp.repeat(lin_w.reshape(-1), 16),
        jnp.broadcast_to(lin_b, (16,)),
    ])
    out_t = _run(xq, tab_flat, wb).reshape(_L, _B, 1)
    return out_t.transpose(1, 0, 2)
